# baseline (speedup 1.0000x reference)
import functools
import math

import jax
import jax.numpy as jnp
from jax import lax
from jax.experimental import pallas as pl
from jax.experimental.pallas import tpu as pltpu
from jax.experimental.pallas import tpu_sc as plsc

NC = 2
NS = 16
LN = 16
CH = 128
D = 128
BLK = 1024


def _sc_mesh():
  return plsc.VectorSubcoreMesh(
      core_axis_name="c", subcore_axis_name="s", num_cores=NC, num_subcores=NS)


def _sc_pass(sd, ew16, tab, npad, epad, width, ch=CH, split=0.5):
  tchunk = epad // (NS * ch)
  n0 = 2 * int(round(split * tchunk / 2.0))
  n1 = tchunk - n0
  assert n0 % 2 == 0 and n1 % 2 == 0 and n0 > 2 and n1 > 2
  rpt = npad // NS

  @functools.partial(
      pl.kernel,
      out_type=jax.ShapeDtypeStruct((NC * npad, width), jnp.float32),
      mesh=_sc_mesh(),
      compiler_params=pltpu.CompilerParams(use_tc_tiling_on_sc=(width == D)),
      scratch_types=[
          pltpu.VMEM((2, ch), jnp.int32),
          pltpu.VMEM((2, ch), jnp.int32),
          pltpu.VMEM((ch,), jnp.int32),
          pltpu.VMEM((ch,), jnp.int32),
          pltpu.VMEM((ch, LN), jnp.float32),
          pltpu.VMEM((ch, LN), jnp.float32),
          pltpu.VMEM((ch, width), jnp.float32),
          pltpu.VMEM((ch, width), jnp.float32),
          pltpu.VMEM_SHARED((npad, width), jnp.float32),
          pltpu.SemaphoreType.DMA,
          pltpu.SemaphoreType.DMA,
          pltpu.SemaphoreType.DMA,
          pltpu.SemaphoreType.DMA,
          pltpu.SemaphoreType.DMA,
          pltpu.SemaphoreType.DMA,
          pltpu.SemaphoreType.DMA,
      ],
  )
  def k(sd_h, ew_h, tab_h, acc_h, sdb0, sdb1, didx0, didx1,
        wvm0, wvm1, rows0, rows1, accsh,
        gsem0, gsem1, wsem0, wsem1, msem0, msem1, ssem):
    cid = lax.axis_index("c")
    sid = lax.axis_index("s")
    nchunk = jnp.where(cid == 0, n0, n1)
    cbase = jnp.where(cid == 0, sid * n0, NS * n0 + sid * n1)
    sdb = (sdb0, sdb1)
    didx = (didx0, didx1)
    wvm = (wvm0, wvm1)
    rows = (rows0, rows1)
    gsem = (gsem0, gsem1)
    wsem = (wsem0, wsem1)
    msem = (msem0, msem1)
    zv = jnp.zeros((LN,), jnp.float32)

    def zrow(i, _):
      for j in range(width // LN):
        rows0[i, pl.ds(j * LN, LN)] = zv
      return 0
    lax.fori_loop(0, ch, zrow, 0)
    for j in range(rpt // ch):
      pltpu.sync_copy(rows0, accsh.at[pl.ds(sid * rpt + j * ch, ch)])

    def meta_start(c, b):
      pltpu.async_copy(sd_h.at[cbase + c], sdb[b], msem[b])

    def meta_wait(c, b):
      pltpu.make_async_copy(sd_h.at[cbase + c], sdb[b], msem[b]).wait()

    def launch(c, b):
      for i in range(ch // LN):
        didx[b][pl.ds(i * LN, LN)] = sdb[b][1, pl.ds(i * LN, LN)]
      pltpu.async_copy(ew_h.at[pl.ds((cbase + c) * ch, ch)],
                       wvm[b], wsem[b])
      pltpu.async_copy(tab_h.at[sdb[b].at[0]], rows[b], gsem[b])

    meta_start(0, 0)
    meta_wait(0, 0)
    launch(0, 0)
    meta_start(1, 1)
    plsc.subcore_barrier()

    def body(g, _):
      for b in range(2):
        c = g * 2 + b
        nb = 1 - b

        @pl.when(c + 1 < nchunk)
        def _():
          @pl.when(c >= 1)
          def _():
            pltpu.make_async_copy(rows[nb], accsh.at[didx[nb]], ssem).wait()
          meta_wait(c + 1, nb)
          launch(c + 1, nb)

        pltpu.make_async_copy(tab_h.at[sdb[b].at[0]], rows[b], gsem[b]).wait()
        pltpu.make_async_copy(ew_h.at[pl.ds((cbase + c) * ch, ch)],
                              wvm[b], wsem[b]).wait()
        @pl.when(c + 2 < nchunk)
        def _():
          meta_start(c + 2, b)

        rb = rows[b]
        wb = wvm[b]

        def scale(e, _):
          wv = wb[e, :]
          for j in range(width // LN):
            rb[e, pl.ds(j * LN, LN)] = rb[e, pl.ds(j * LN, LN)] * wv
          return 0
        lax.fori_loop(0, ch, scale, 0, unroll=4)
        pltpu.async_copy(rows[b], accsh.at[didx[b]], ssem, add=True)
      return 0
    lax.fori_loop(0, nchunk // 2, body, 0)

    pltpu.make_async_copy(rows[0], accsh.at[didx[0]], ssem).wait()
    pltpu.make_async_copy(rows[1], accsh.at[didx[1]], ssem).wait()
    plsc.subcore_barrier()
    pltpu.sync_copy(accsh.at[pl.ds(sid * rpt, rpt)],
                    acc_h.at[pl.ds(cid * npad + sid * rpt, rpt)])

  return k(sd, ew16, tab)


def _tc_a(xin, mcol, kpcol, degtab, W, first, npad):
  grid = npad // BLK

  def body(x_ref, m_ref, kp_ref, deg_ref, w_ref, h2s_ref, dis_ref):
    x = x_ref[...]
    if not first:
      x = jnp.maximum(x * m_ref[...], 0.0)
    degraw = jnp.sum(deg_ref[...][:, :, :1], axis=0)
    deg = kp_ref[...] * (degraw + 1.0)
    safe = jnp.where(deg > 0, deg, 1.0)
    dis = jnp.where(deg > 0, 1.0 / jnp.sqrt(safe), 0.0)
    h = jnp.dot(x, w_ref[...], preferred_element_type=jnp.float32)
    h2s_ref[...] = h * dis
    dis_ref[...] = dis

  return pl.pallas_call(
      body,
      grid=(grid,),
      in_specs=[
          pl.BlockSpec((BLK, D), lambda i: (i, 0)),
          pl.BlockSpec((BLK, 1), lambda i: (i, 0)),
          pl.BlockSpec((BLK, 1), lambda i: (i, 0)),
          pl.BlockSpec((NC, BLK, LN), lambda i: (0, i, 0)),
          pl.BlockSpec((D, D), lambda i: (0, 0)),
      ],
      out_specs=[
          pl.BlockSpec((BLK, D), lambda i: (i, 0)),
          pl.BlockSpec((BLK, 1), lambda i: (i, 0)),
      ],
      out_shape=[
          jax.ShapeDtypeStruct((npad, D), jnp.float32),
          jax.ShapeDtypeStruct((npad, 1), jnp.float32),
      ],
  )(xin, mcol, kpcol, degtab, W)


def _tc_b1(aggparts, h2s, dis_col, bvec, pcol, npad):
  grid = npad // BLK

  def body(agg_ref, h_ref, dis_ref, b_ref, p_ref, o_ref, z_ref):
    s = h_ref[...] + agg_ref[0] + agg_ref[1]
    o = b_ref[...] + dis_ref[...] * s
    o_ref[...] = o
    pv = p_ref[...]
    n2 = jnp.sum(pv * pv)
    z_ref[...] = jnp.dot(o, pv, preferred_element_type=jnp.float32) / jnp.sqrt(n2)

  return pl.pallas_call(
      body,
      grid=(grid,),
      in_specs=[
          pl.BlockSpec((NC, BLK, D), lambda i: (0, i, 0)),
          pl.BlockSpec((BLK, D), lambda i: (i, 0)),
          pl.BlockSpec((BLK, 1), lambda i: (i, 0)),
          pl.BlockSpec((1, D), lambda i: (0, 0)),
          pl.BlockSpec((D, 1), lambda i: (0, 0)),
      ],
      out_specs=[
          pl.BlockSpec((BLK, D), lambda i: (i, 0)),
          pl.BlockSpec((BLK, 1), lambda i: (i, 0)),
      ],
      out_shape=[
          jax.ShapeDtypeStruct((npad, D), jnp.float32),
          jax.ShapeDtypeStruct((npad, 1), jnp.float32),
      ],
  )(aggparts, h2s, dis_col, bvec, pcol)


def _tc_b2(z_pack, kprev_pack, kth, npad):
  rows = npad // 128

  def body(z_ref, kp_ref, m_ref, keep_ref):
    z = z_ref[...]
    elig = kp_ref[...] > 0.0
    s = 1.0 / (1.0 + jnp.exp(-z))
    svals = jnp.where(elig, s, -1.0)
    key = lax.bitcast_convert_type(svals, jnp.int32)
    kk = jnp.int32(kth)

    def bit_step(i, t):
      t2 = t | (jnp.int32(1) << (30 - i))
      cnt = jnp.sum((key >= t2).astype(jnp.int32))
      return jnp.where(cnt >= kk, t2, t)
    tstar = lax.fori_loop(0, 31, bit_step, jnp.int32(0))

    gt = key > tstar
    eq = key == tstar
    rem = kk - jnp.sum(gt.astype(jnp.int32))
    idx = (lax.broadcasted_iota(jnp.int32, z.shape, 0) * 128
           + lax.broadcasted_iota(jnp.int32, z.shape, 1))

    def bit_step2(i, xthr):
      x2 = xthr | (jnp.int32(1) << (13 - i))
      cnt = jnp.sum((eq & (idx < x2)).astype(jnp.int32))
      return jnp.where(cnt < rem, x2, xthr)
    xb = lax.fori_loop(0, 14, bit_step2, jnp.int32(0))

    keep = (gt | (eq & (idx <= xb))).astype(jnp.float32)
    keep_ref[...] = keep
    m_ref[...] = keep * s

  return pl.pallas_call(
      body,
      in_specs=[
          pl.BlockSpec((rows, 128), lambda: (0, 0)),
          pl.BlockSpec((rows, 128), lambda: (0, 0)),
      ],
      out_specs=[
          pl.BlockSpec((rows, 128), lambda: (0, 0)),
          pl.BlockSpec((rows, 128), lambda: (0, 0)),
      ],
      out_shape=[
          jax.ShapeDtypeStruct((rows, 128), jnp.float32),
          jax.ShapeDtypeStruct((rows, 128), jnp.float32),
      ],
  )(z_pack, kprev_pack)


def _tc_readout(o, mcol, Wo, bo, cnt, npad):
  def body(o_ref, m_ref, wo_ref, bo_ref, out_ref, xc_ref):
    x4 = jnp.maximum(o_ref[...] * m_ref[...], 0.0)
    gmax = jnp.max(x4, axis=0, keepdims=True)
    gmean = jnp.sum(x4, axis=0, keepdims=True) / cnt
    xc = jnp.concatenate([gmax, gmean], axis=1)
    xc_ref[...] = xc
    val = jnp.dot(xc, wo_ref[...], preferred_element_type=jnp.float32) + bo_ref[...]
    out_ref[...] = 1.0 / (1.0 + jnp.exp(-val))

  return pl.pallas_call(
      body,
      in_specs=[
          pl.BlockSpec((npad, D), lambda: (0, 0)),
          pl.BlockSpec((npad, 1), lambda: (0, 0)),
          pl.BlockSpec((2 * D, 1), lambda: (0, 0)),
          pl.BlockSpec((1, 1), lambda: (0, 0)),
      ],
      out_specs=[
          pl.BlockSpec((1, 1), lambda: (0, 0)),
          pl.BlockSpec((1, 2 * D), lambda: (0, 0)),
      ],
      out_shape=[
          jax.ShapeDtypeStruct((1, 1), jnp.float32),
          jax.ShapeDtypeStruct((1, 2 * D), jnp.float32),
      ],
  )(o, mcol, Wo, bo)


def kernel(x, edge_index, edge_weight, batch_index,
           W1, b1, p1, W2, b2, p2, W3, b3, p3, Wo, bo):
  n, d = x.shape
  e = edge_index.shape[1]
  assert d == D
  npad = ((n + NS * CH - 1) // (NS * CH)) * (NS * CH)
  estep = NC * NS * CH * 2
  epad = ((e + estep - 1) // estep) * estep

  src = jnp.concatenate([edge_index[0], jnp.zeros((epad - e,), jnp.int32)])
  dst = jnp.concatenate([edge_index[1], jnp.zeros((epad - e,), jnp.int32)])
  sd = jnp.stack([src.reshape(epad // CH, CH), dst.reshape(epad // CH, CH)], axis=1)
  sd64 = jnp.stack([src.reshape(epad // 64, 64), dst.reshape(epad // 64, 64)], axis=1)
  ew = jnp.concatenate([edge_weight, jnp.zeros((epad - e,), jnp.float32)])
  ew16 = ew.reshape(epad, 1) * jnp.ones((1, LN), jnp.float32)
  xp = jnp.concatenate([x, jnp.zeros((npad - n, d), jnp.float32)])

  keep = jnp.concatenate([jnp.ones((n,), jnp.float32),
                          jnp.zeros((npad - n,), jnp.float32)])

  ks = []
  kk = n
  for _ in range(3):
    kk = int(math.ceil(0.8 * kk))
    ks.append(kk)

  layers = [(W1, b1, p1, ks[0]), (W2, b2, p2, ks[1]), (W3, b3, p3, ks[2])]
  o = xp
  mcol = jnp.zeros((npad, 1), jnp.float32)
  for li, (W, b, p, kth) in enumerate(layers):
    keep16 = keep.reshape(npad, 1) * jnp.ones((1, LN), jnp.float32)
    degtab = _sc_pass(sd, ew16, keep16, npad, epad, LN).reshape(NC, npad, LN)
    h2s, dis_col = _tc_a(o, mcol, keep.reshape(npad, 1), degtab, W,
                         first=(li == 0), npad=npad)
    aggparts = _sc_pass(sd64, ew16, h2s, npad, epad, D, ch=64, split=0.28).reshape(NC, npad, D)
    o, z_col = _tc_b1(aggparts, h2s, dis_col, b.reshape(1, D), p.reshape(D, 1), npad)
    m_pack, keep_pack = _tc_b2(z_col.reshape(npad // 128, 128),
                               keep.reshape(npad // 128, 128), kth, npad)
    mcol = m_pack.reshape(npad, 1)
    keep = keep_pack.reshape(npad)

  out, xc = _tc_readout(o, mcol, Wo.reshape(2 * D, 1), bo.reshape(1, 1),
                        float(ks[2]), npad)
  return out, xc


# speedup vs baseline: 1.1134x; 1.1134x over previous
import functools
import math

import jax
import jax.numpy as jnp
from jax import lax
from jax.experimental import pallas as pl
from jax.experimental.pallas import tpu as pltpu
from jax.experimental.pallas import tpu_sc as plsc

NC = 2
NS = 16
LN = 16
CH = 128
D = 128
BLK = 1024


def _sc_mesh():
  return plsc.VectorSubcoreMesh(
      core_axis_name="c", subcore_axis_name="s", num_cores=NC, num_subcores=NS)


def _sc_pass(sd, ew16, tab, npad, epad, width, ch=CH, split=0.5):
  tchunk = epad // (NS * ch)
  n0 = 2 * int(round(split * tchunk / 2.0))
  n1 = tchunk - n0
  assert n0 % 2 == 0 and n1 % 2 == 0 and n0 > 2 and n1 > 2
  rpt = npad // NS

  @functools.partial(
      pl.kernel,
      out_type=jax.ShapeDtypeStruct((NC * npad, width), jnp.float32),
      mesh=_sc_mesh(),
      compiler_params=pltpu.CompilerParams(use_tc_tiling_on_sc=(width == D)),
      scratch_types=[
          pltpu.VMEM((2, ch), jnp.int32),
          pltpu.VMEM((2, ch), jnp.int32),
          pltpu.VMEM((ch,), jnp.int32),
          pltpu.VMEM((ch,), jnp.int32),
          pltpu.VMEM((ch, LN), jnp.float32),
          pltpu.VMEM((ch, LN), jnp.float32),
          pltpu.VMEM((ch, width), jnp.float32),
          pltpu.VMEM((ch, width), jnp.float32),
          pltpu.VMEM_SHARED((npad, width), jnp.float32),
          pltpu.SemaphoreType.DMA,
          pltpu.SemaphoreType.DMA,
          pltpu.SemaphoreType.DMA,
          pltpu.SemaphoreType.DMA,
          pltpu.SemaphoreType.DMA,
          pltpu.SemaphoreType.DMA,
          pltpu.SemaphoreType.DMA,
      ],
  )
  def k(sd_h, ew_h, tab_h, acc_h, sdb0, sdb1, didx0, didx1,
        wvm0, wvm1, rows0, rows1, accsh,
        gsem0, gsem1, wsem0, wsem1, msem0, msem1, ssem):
    cid = lax.axis_index("c")
    sid = lax.axis_index("s")
    nchunk = jnp.where(cid == 0, n0, n1)
    cbase = jnp.where(cid == 0, sid * n0, NS * n0 + sid * n1)
    sdb = (sdb0, sdb1)
    didx = (didx0, didx1)
    wvm = (wvm0, wvm1)
    rows = (rows0, rows1)
    gsem = (gsem0, gsem1)
    wsem = (wsem0, wsem1)
    msem = (msem0, msem1)
    zv = jnp.zeros((LN,), jnp.float32)

    def zrow(i, _):
      for j in range(width // LN):
        rows0[i, pl.ds(j * LN, LN)] = zv
      return 0
    lax.fori_loop(0, ch, zrow, 0)
    for j in range(rpt // ch):
      pltpu.sync_copy(rows0, accsh.at[pl.ds(sid * rpt + j * ch, ch)])

    def meta_start(c, b):
      pltpu.async_copy(sd_h.at[cbase + c], sdb[b], msem[b])

    def meta_wait(c, b):
      pltpu.make_async_copy(sd_h.at[cbase + c], sdb[b], msem[b]).wait()

    def launch(c, b):
      for i in range(ch // LN):
        didx[b][pl.ds(i * LN, LN)] = sdb[b][1, pl.ds(i * LN, LN)]
      pltpu.async_copy(ew_h.at[pl.ds((cbase + c) * ch, ch)],
                       wvm[b], wsem[b])
      pltpu.async_copy(tab_h.at[sdb[b].at[0]], rows[b], gsem[b])

    meta_start(0, 0)
    meta_wait(0, 0)
    launch(0, 0)
    meta_start(1, 1)
    plsc.subcore_barrier()

    def body(g, _):
      for b in range(2):
        c = g * 2 + b
        nb = 1 - b

        @pl.when(c + 1 < nchunk)
        def _():
          @pl.when(c >= 1)
          def _():
            pltpu.make_async_copy(rows[nb], accsh.at[didx[nb]], ssem).wait()
          meta_wait(c + 1, nb)
          launch(c + 1, nb)

        pltpu.make_async_copy(tab_h.at[sdb[b].at[0]], rows[b], gsem[b]).wait()
        pltpu.make_async_copy(ew_h.at[pl.ds((cbase + c) * ch, ch)],
                              wvm[b], wsem[b]).wait()
        @pl.when(c + 2 < nchunk)
        def _():
          meta_start(c + 2, b)

        rb = rows[b]
        wb = wvm[b]

        def scale(e, _):
          wv = wb[e, :]
          for j in range(width // LN):
            rb[e, pl.ds(j * LN, LN)] = rb[e, pl.ds(j * LN, LN)] * wv
          return 0
        lax.fori_loop(0, ch, scale, 0, unroll=4)
        pltpu.async_copy(rows[b], accsh.at[didx[b]], ssem, add=True)
      return 0
    lax.fori_loop(0, nchunk // 2, body, 0)

    pltpu.make_async_copy(rows[0], accsh.at[didx[0]], ssem).wait()
    pltpu.make_async_copy(rows[1], accsh.at[didx[1]], ssem).wait()
    plsc.subcore_barrier()
    pltpu.sync_copy(accsh.at[pl.ds(sid * rpt, rpt)],
                    acc_h.at[pl.ds(cid * npad + sid * rpt, rpt)])

  return k(sd, ew16, tab)


def _tc_a(xin, mcol, kpcol, degtab, W, first, npad):
  grid = npad // BLK

  def body(x_ref, m_ref, kp_ref, deg_ref, w_ref, h2s_ref, dis_ref):
    x = x_ref[...]
    if not first:
      x = jnp.maximum(x * m_ref[...], 0.0)
    degraw = jnp.sum(deg_ref[...][:, :, :1], axis=0)
    deg = kp_ref[...] * (degraw + 1.0)
    safe = jnp.where(deg > 0, deg, 1.0)
    dis = jnp.where(deg > 0, 1.0 / jnp.sqrt(safe), 0.0)
    h = jnp.dot(x, w_ref[...], preferred_element_type=jnp.float32)
    h2s_ref[...] = h * dis
    dis_ref[...] = dis

  return pl.pallas_call(
      body,
      grid=(grid,),
      in_specs=[
          pl.BlockSpec((BLK, D), lambda i: (i, 0)),
          pl.BlockSpec((BLK, 1), lambda i: (i, 0)),
          pl.BlockSpec((BLK, 1), lambda i: (i, 0)),
          pl.BlockSpec((NC, BLK, LN), lambda i: (0, i, 0)),
          pl.BlockSpec((D, D), lambda i: (0, 0)),
      ],
      out_specs=[
          pl.BlockSpec((BLK, D), lambda i: (i, 0)),
          pl.BlockSpec((BLK, 1), lambda i: (i, 0)),
      ],
      out_shape=[
          jax.ShapeDtypeStruct((npad, D), jnp.float32),
          jax.ShapeDtypeStruct((npad, 1), jnp.float32),
      ],
  )(xin, mcol, kpcol, degtab, W)


def _tc_b1(aggparts, h2s, dis_col, bvec, pcol, npad):
  grid = npad // BLK

  def body(agg_ref, h_ref, dis_ref, b_ref, p_ref, o_ref, z_ref):
    s = h_ref[...] + agg_ref[0] + agg_ref[1]
    o = b_ref[...] + dis_ref[...] * s
    o_ref[...] = o
    pv = p_ref[...]
    n2 = jnp.sum(pv * pv)
    z_ref[...] = jnp.dot(o, pv, preferred_element_type=jnp.float32) / jnp.sqrt(n2)

  return pl.pallas_call(
      body,
      grid=(grid,),
      in_specs=[
          pl.BlockSpec((NC, BLK, D), lambda i: (0, i, 0)),
          pl.BlockSpec((BLK, D), lambda i: (i, 0)),
          pl.BlockSpec((BLK, 1), lambda i: (i, 0)),
          pl.BlockSpec((1, D), lambda i: (0, 0)),
          pl.BlockSpec((D, 1), lambda i: (0, 0)),
      ],
      out_specs=[
          pl.BlockSpec((BLK, D), lambda i: (i, 0)),
          pl.BlockSpec((BLK, 1), lambda i: (i, 0)),
      ],
      out_shape=[
          jax.ShapeDtypeStruct((npad, D), jnp.float32),
          jax.ShapeDtypeStruct((npad, 1), jnp.float32),
      ],
  )(aggparts, h2s, dis_col, bvec, pcol)


def _tc_b2(z_pack, kprev_pack, kth, npad):
  rows = npad // 128

  def body(z_ref, kp_ref, m_ref, keep_ref):
    z = z_ref[...]
    elig = kp_ref[...] > 0.0
    s = 1.0 / (1.0 + jnp.exp(-z))
    svals = jnp.where(elig, s, -1.0)
    key = lax.bitcast_convert_type(svals, jnp.int32)
    kk = jnp.int32(kth)

    def bit_step(i, t):
      t2 = t | (jnp.int32(1) << (30 - i))
      cnt = jnp.sum((key >= t2).astype(jnp.int32))
      return jnp.where(cnt >= kk, t2, t)
    tstar = lax.fori_loop(0, 31, bit_step, jnp.int32(0))

    gt = key > tstar
    eq = key == tstar
    rem = kk - jnp.sum(gt.astype(jnp.int32))
    idx = (lax.broadcasted_iota(jnp.int32, z.shape, 0) * 128
           + lax.broadcasted_iota(jnp.int32, z.shape, 1))

    def bit_step2(i, xthr):
      x2 = xthr | (jnp.int32(1) << (13 - i))
      cnt = jnp.sum((eq & (idx < x2)).astype(jnp.int32))
      return jnp.where(cnt < rem, x2, xthr)
    xb = lax.fori_loop(0, 14, bit_step2, jnp.int32(0))

    keep = (gt | (eq & (idx <= xb))).astype(jnp.float32)
    keep_ref[...] = keep
    m_ref[...] = keep * s

  return pl.pallas_call(
      body,
      in_specs=[
          pl.BlockSpec((rows, 128), lambda: (0, 0)),
          pl.BlockSpec((rows, 128), lambda: (0, 0)),
      ],
      out_specs=[
          pl.BlockSpec((rows, 128), lambda: (0, 0)),
          pl.BlockSpec((rows, 128), lambda: (0, 0)),
      ],
      out_shape=[
          jax.ShapeDtypeStruct((rows, 128), jnp.float32),
          jax.ShapeDtypeStruct((rows, 128), jnp.float32),
      ],
  )(z_pack, kprev_pack)


def _tc_readout(o, mcol, Wo, bo, cnt, npad):
  def body(o_ref, m_ref, wo_ref, bo_ref, out_ref, xc_ref):
    x4 = jnp.maximum(o_ref[...] * m_ref[...], 0.0)
    gmax = jnp.max(x4, axis=0, keepdims=True)
    gmean = jnp.sum(x4, axis=0, keepdims=True) / cnt
    xc = jnp.concatenate([gmax, gmean], axis=1)
    xc_ref[...] = xc
    val = jnp.dot(xc, wo_ref[...], preferred_element_type=jnp.float32) + bo_ref[...]
    out_ref[...] = 1.0 / (1.0 + jnp.exp(-val))

  return pl.pallas_call(
      body,
      in_specs=[
          pl.BlockSpec((npad, D), lambda: (0, 0)),
          pl.BlockSpec((npad, 1), lambda: (0, 0)),
          pl.BlockSpec((2 * D, 1), lambda: (0, 0)),
          pl.BlockSpec((1, 1), lambda: (0, 0)),
      ],
      out_specs=[
          pl.BlockSpec((1, 1), lambda: (0, 0)),
          pl.BlockSpec((1, 2 * D), lambda: (0, 0)),
      ],
      out_shape=[
          jax.ShapeDtypeStruct((1, 1), jnp.float32),
          jax.ShapeDtypeStruct((1, 2 * D), jnp.float32),
      ],
  )(o, mcol, Wo, bo)


def kernel(x, edge_index, edge_weight, batch_index,
           W1, b1, p1, W2, b2, p2, W3, b3, p3, Wo, bo):
  n, d = x.shape
  e = edge_index.shape[1]
  assert d == D
  npad = ((n + NS * CH - 1) // (NS * CH)) * (NS * CH)
  estep = NC * NS * CH * 2
  epad = ((e + estep - 1) // estep) * estep

  src = jnp.concatenate([edge_index[0], jnp.zeros((epad - e,), jnp.int32)])
  dst = jnp.concatenate([edge_index[1], jnp.zeros((epad - e,), jnp.int32)])
  sd = jnp.stack([src.reshape(epad // CH, CH), dst.reshape(epad // CH, CH)], axis=1)
  sd64 = jnp.stack([src.reshape(epad // 64, 64), dst.reshape(epad // 64, 64)], axis=1)
  ew = jnp.concatenate([edge_weight, jnp.zeros((epad - e,), jnp.float32)])
  ew16 = ew.reshape(epad, 1) * jnp.ones((1, LN), jnp.float32)
  xp = jnp.concatenate([x, jnp.zeros((npad - n, d), jnp.float32)])

  keep = jnp.concatenate([jnp.ones((n,), jnp.float32),
                          jnp.zeros((npad - n,), jnp.float32)])

  ks = []
  kk = n
  for _ in range(3):
    kk = int(math.ceil(0.8 * kk))
    ks.append(kk)

  layers = [(W1, b1, p1, ks[0]), (W2, b2, p2, ks[1]), (W3, b3, p3, ks[2])]
  o = xp
  mcol = jnp.zeros((npad, 1), jnp.float32)
  for li, (W, b, p, kth) in enumerate(layers):
    keep16 = keep.reshape(npad, 1) * jnp.ones((1, LN), jnp.float32)
    degtab = _sc_pass(sd, ew16, keep16, npad, epad, LN).reshape(NC, npad, LN)
    h2s, dis_col = _tc_a(o, mcol, keep.reshape(npad, 1), degtab, W,
                         first=(li == 0), npad=npad)
    aggparts = _sc_pass(sd64, ew16, h2s, npad, epad, D, ch=64, split=0.72).reshape(NC, npad, D)
    o, z_col = _tc_b1(aggparts, h2s, dis_col, b.reshape(1, D), p.reshape(D, 1), npad)
    m_pack, keep_pack = _tc_b2(z_col.reshape(npad // 128, 128),
                               keep.reshape(npad // 128, 128), kth, npad)
    mcol = m_pack.reshape(npad, 1)
    keep = keep_pack.reshape(npad)

  out, xc = _tc_readout(o, mcol, Wo.reshape(2 * D, 1), bo.reshape(1, 1),
                        float(ks[2]), npad)
  return out, xc


# speedup vs baseline: 1.1267x; 1.0119x over previous
import functools
import math

import jax
import jax.numpy as jnp
from jax import lax
from jax.experimental import pallas as pl
from jax.experimental.pallas import tpu as pltpu
from jax.experimental.pallas import tpu_sc as plsc

NC = 2
NS = 16
LN = 16
CH = 128
D = 128
BLK = 1024


def _sc_mesh():
  return plsc.VectorSubcoreMesh(
      core_axis_name="c", subcore_axis_name="s", num_cores=NC, num_subcores=NS)


def _sc_pass(sd, ew16, tab, npad, epad, width, ch=CH, split=0.5):
  tchunk = epad // (NS * ch)
  n0 = 2 * int(round(split * tchunk / 2.0))
  n1 = tchunk - n0
  assert n0 % 2 == 0 and n1 % 2 == 0 and n0 > 2 and n1 > 2
  rpt = npad // NS

  @functools.partial(
      pl.kernel,
      out_type=jax.ShapeDtypeStruct((NC * npad, width), jnp.float32),
      mesh=_sc_mesh(),
      compiler_params=pltpu.CompilerParams(use_tc_tiling_on_sc=(width == D)),
      scratch_types=[
          pltpu.VMEM((2, ch), jnp.int32),
          pltpu.VMEM((2, ch), jnp.int32),
          pltpu.VMEM((ch,), jnp.int32),
          pltpu.VMEM((ch,), jnp.int32),
          pltpu.VMEM((ch, LN), jnp.float32),
          pltpu.VMEM((ch, LN), jnp.float32),
          pltpu.VMEM((ch, width), jnp.float32),
          pltpu.VMEM((ch, width), jnp.float32),
          pltpu.VMEM_SHARED((npad, width), jnp.float32),
          pltpu.SemaphoreType.DMA,
          pltpu.SemaphoreType.DMA,
          pltpu.SemaphoreType.DMA,
          pltpu.SemaphoreType.DMA,
          pltpu.SemaphoreType.DMA,
          pltpu.SemaphoreType.DMA,
          pltpu.SemaphoreType.DMA,
      ],
  )
  def k(sd_h, ew_h, tab_h, acc_h, sdb0, sdb1, didx0, didx1,
        wvm0, wvm1, rows0, rows1, accsh,
        gsem0, gsem1, wsem0, wsem1, msem0, msem1, ssem):
    cid = lax.axis_index("c")
    sid = lax.axis_index("s")
    nchunk = jnp.where(cid == 0, n0, n1)
    cbase = jnp.where(cid == 0, sid * n0, NS * n0 + sid * n1)
    sdb = (sdb0, sdb1)
    didx = (didx0, didx1)
    wvm = (wvm0, wvm1)
    rows = (rows0, rows1)
    gsem = (gsem0, gsem1)
    wsem = (wsem0, wsem1)
    msem = (msem0, msem1)
    zv = jnp.zeros((LN,), jnp.float32)

    def zrow(i, _):
      for j in range(width // LN):
        rows0[i, pl.ds(j * LN, LN)] = zv
      return 0
    lax.fori_loop(0, ch, zrow, 0)
    for j in range(rpt // ch):
      pltpu.sync_copy(rows0, accsh.at[pl.ds(sid * rpt + j * ch, ch)])

    def meta_start(c, b):
      pltpu.async_copy(sd_h.at[cbase + c], sdb[b], msem[b])

    def meta_wait(c, b):
      pltpu.make_async_copy(sd_h.at[cbase + c], sdb[b], msem[b]).wait()

    def launch(c, b):
      for i in range(ch // LN):
        didx[b][pl.ds(i * LN, LN)] = sdb[b][1, pl.ds(i * LN, LN)]
      pltpu.async_copy(ew_h.at[pl.ds((cbase + c) * ch, ch)],
                       wvm[b], wsem[b])
      pltpu.async_copy(tab_h.at[sdb[b].at[0]], rows[b], gsem[b])

    meta_start(0, 0)
    meta_wait(0, 0)
    launch(0, 0)
    meta_start(1, 1)
    plsc.subcore_barrier()

    def body(g, _):
      for b in range(2):
        c = g * 2 + b
        nb = 1 - b

        @pl.when(c + 1 < nchunk)
        def _():
          @pl.when(c >= 1)
          def _():
            pltpu.make_async_copy(rows[nb], accsh.at[didx[nb]], ssem).wait()
          meta_wait(c + 1, nb)
          launch(c + 1, nb)

        pltpu.make_async_copy(tab_h.at[sdb[b].at[0]], rows[b], gsem[b]).wait()
        pltpu.make_async_copy(ew_h.at[pl.ds((cbase + c) * ch, ch)],
                              wvm[b], wsem[b]).wait()
        @pl.when(c + 2 < nchunk)
        def _():
          meta_start(c + 2, b)

        rb = rows[b]
        wb = wvm[b]

        def scale(e, _):
          wv = wb[e, :]
          for j in range(width // LN):
            rb[e, pl.ds(j * LN, LN)] = rb[e, pl.ds(j * LN, LN)] * wv
          return 0
        lax.fori_loop(0, ch, scale, 0, unroll=4)
        pltpu.async_copy(rows[b], accsh.at[didx[b]], ssem, add=True)
      return 0
    lax.fori_loop(0, nchunk // 2, body, 0)

    pltpu.make_async_copy(rows[0], accsh.at[didx[0]], ssem).wait()
    pltpu.make_async_copy(rows[1], accsh.at[didx[1]], ssem).wait()
    plsc.subcore_barrier()
    pltpu.sync_copy(accsh.at[pl.ds(sid * rpt, rpt)],
                    acc_h.at[pl.ds(cid * npad + sid * rpt, rpt)])

  return k(sd, ew16, tab)


def _tc_a(xin, mcol, kpcol, degtab, W, first, npad):
  grid = npad // BLK

  def body(x_ref, m_ref, kp_ref, deg_ref, w_ref, h2s_ref, dis_ref):
    x = x_ref[...]
    if not first:
      x = jnp.maximum(x * m_ref[...], 0.0)
    degraw = jnp.sum(deg_ref[...][:, :, :1], axis=0)
    deg = kp_ref[...] * (degraw + 1.0)
    safe = jnp.where(deg > 0, deg, 1.0)
    dis = jnp.where(deg > 0, 1.0 / jnp.sqrt(safe), 0.0)
    h = jnp.dot(x, w_ref[...], preferred_element_type=jnp.float32)
    h2s_ref[...] = h * dis
    dis_ref[...] = dis

  return pl.pallas_call(
      body,
      grid=(grid,),
      in_specs=[
          pl.BlockSpec((BLK, D), lambda i: (i, 0)),
          pl.BlockSpec((BLK, 1), lambda i: (i, 0)),
          pl.BlockSpec((BLK, 1), lambda i: (i, 0)),
          pl.BlockSpec((NC, BLK, LN), lambda i: (0, i, 0)),
          pl.BlockSpec((D, D), lambda i: (0, 0)),
      ],
      out_specs=[
          pl.BlockSpec((BLK, D), lambda i: (i, 0)),
          pl.BlockSpec((BLK, 1), lambda i: (i, 0)),
      ],
      out_shape=[
          jax.ShapeDtypeStruct((npad, D), jnp.float32),
          jax.ShapeDtypeStruct((npad, 1), jnp.float32),
      ],
  )(xin, mcol, kpcol, degtab, W)


def _tc_b1(aggparts, h2s, dis_col, bvec, pcol, npad):
  grid = npad // BLK

  def body(agg_ref, h_ref, dis_ref, b_ref, p_ref, o_ref, z_ref):
    s = h_ref[...] + agg_ref[0] + agg_ref[1]
    o = b_ref[...] + dis_ref[...] * s
    o_ref[...] = o
    pv = p_ref[...]
    n2 = jnp.sum(pv * pv)
    z_ref[...] = jnp.dot(o, pv, preferred_element_type=jnp.float32) / jnp.sqrt(n2)

  return pl.pallas_call(
      body,
      grid=(grid,),
      in_specs=[
          pl.BlockSpec((NC, BLK, D), lambda i: (0, i, 0)),
          pl.BlockSpec((BLK, D), lambda i: (i, 0)),
          pl.BlockSpec((BLK, 1), lambda i: (i, 0)),
          pl.BlockSpec((1, D), lambda i: (0, 0)),
          pl.BlockSpec((D, 1), lambda i: (0, 0)),
      ],
      out_specs=[
          pl.BlockSpec((BLK, D), lambda i: (i, 0)),
          pl.BlockSpec((BLK, 1), lambda i: (i, 0)),
      ],
      out_shape=[
          jax.ShapeDtypeStruct((npad, D), jnp.float32),
          jax.ShapeDtypeStruct((npad, 1), jnp.float32),
      ],
  )(aggparts, h2s, dis_col, bvec, pcol)


def _tc_b2(z_pack, kprev_pack, kth, npad):
  rows = npad // 128

  def body(z_ref, kp_ref, m_ref, keep_ref):
    z = z_ref[...]
    elig = kp_ref[...] > 0.0
    s = 1.0 / (1.0 + jnp.exp(-z))
    svals = jnp.where(elig, s, -1.0)
    key = lax.bitcast_convert_type(svals, jnp.int32)
    kk = jnp.int32(kth)

    def bit_step(i, t):
      t2 = t | (jnp.int32(1) << (30 - i))
      cnt = jnp.sum((key >= t2).astype(jnp.int32))
      return jnp.where(cnt >= kk, t2, t)
    tstar = lax.fori_loop(0, 31, bit_step, jnp.int32(0))

    gt = key > tstar
    eq = key == tstar
    rem = kk - jnp.sum(gt.astype(jnp.int32))
    idx = (lax.broadcasted_iota(jnp.int32, z.shape, 0) * 128
           + lax.broadcasted_iota(jnp.int32, z.shape, 1))

    def bit_step2(i, xthr):
      x2 = xthr | (jnp.int32(1) << (13 - i))
      cnt = jnp.sum((eq & (idx < x2)).astype(jnp.int32))
      return jnp.where(cnt < rem, x2, xthr)
    xb = lax.fori_loop(0, 14, bit_step2, jnp.int32(0))

    keep = (gt | (eq & (idx <= xb))).astype(jnp.float32)
    keep_ref[...] = keep
    m_ref[...] = keep * s

  return pl.pallas_call(
      body,
      in_specs=[
          pl.BlockSpec((rows, 128), lambda: (0, 0)),
          pl.BlockSpec((rows, 128), lambda: (0, 0)),
      ],
      out_specs=[
          pl.BlockSpec((rows, 128), lambda: (0, 0)),
          pl.BlockSpec((rows, 128), lambda: (0, 0)),
      ],
      out_shape=[
          jax.ShapeDtypeStruct((rows, 128), jnp.float32),
          jax.ShapeDtypeStruct((rows, 128), jnp.float32),
      ],
  )(z_pack, kprev_pack)


def _tc_readout(o, mcol, Wo, bo, cnt, npad):
  def body(o_ref, m_ref, wo_ref, bo_ref, out_ref, xc_ref):
    x4 = jnp.maximum(o_ref[...] * m_ref[...], 0.0)
    gmax = jnp.max(x4, axis=0, keepdims=True)
    gmean = jnp.sum(x4, axis=0, keepdims=True) / cnt
    xc = jnp.concatenate([gmax, gmean], axis=1)
    xc_ref[...] = xc
    val = jnp.dot(xc, wo_ref[...], preferred_element_type=jnp.float32) + bo_ref[...]
    out_ref[...] = 1.0 / (1.0 + jnp.exp(-val))

  return pl.pallas_call(
      body,
      in_specs=[
          pl.BlockSpec((npad, D), lambda: (0, 0)),
          pl.BlockSpec((npad, 1), lambda: (0, 0)),
          pl.BlockSpec((2 * D, 1), lambda: (0, 0)),
          pl.BlockSpec((1, 1), lambda: (0, 0)),
      ],
      out_specs=[
          pl.BlockSpec((1, 1), lambda: (0, 0)),
          pl.BlockSpec((1, 2 * D), lambda: (0, 0)),
      ],
      out_shape=[
          jax.ShapeDtypeStruct((1, 1), jnp.float32),
          jax.ShapeDtypeStruct((1, 2 * D), jnp.float32),
      ],
  )(o, mcol, Wo, bo)


def kernel(x, edge_index, edge_weight, batch_index,
           W1, b1, p1, W2, b2, p2, W3, b3, p3, Wo, bo):
  n, d = x.shape
  e = edge_index.shape[1]
  assert d == D
  npad = ((n + NS * CH - 1) // (NS * CH)) * (NS * CH)
  estep = NC * NS * CH * 2
  epad = ((e + estep - 1) // estep) * estep

  src = jnp.concatenate([edge_index[0], jnp.zeros((epad - e,), jnp.int32)])
  dst = jnp.concatenate([edge_index[1], jnp.zeros((epad - e,), jnp.int32)])
  sd = jnp.stack([src.reshape(epad // CH, CH), dst.reshape(epad // CH, CH)], axis=1)
  sd64 = jnp.stack([src.reshape(epad // 64, 64), dst.reshape(epad // 64, 64)], axis=1)
  ew = jnp.concatenate([edge_weight, jnp.zeros((epad - e,), jnp.float32)])
  ew16 = ew.reshape(epad, 1) * jnp.ones((1, LN), jnp.float32)
  xp = jnp.concatenate([x, jnp.zeros((npad - n, d), jnp.float32)])

  keep = jnp.concatenate([jnp.ones((n,), jnp.float32),
                          jnp.zeros((npad - n,), jnp.float32)])

  ks = []
  kk = n
  for _ in range(3):
    kk = int(math.ceil(0.8 * kk))
    ks.append(kk)

  layers = [(W1, b1, p1, ks[0]), (W2, b2, p2, ks[1]), (W3, b3, p3, ks[2])]
  o = xp
  mcol = jnp.zeros((npad, 1), jnp.float32)
  for li, (W, b, p, kth) in enumerate(layers):
    keep16 = keep.reshape(npad, 1) * jnp.ones((1, LN), jnp.float32)
    degtab = _sc_pass(sd, ew16, keep16, npad, epad, LN).reshape(NC, npad, LN)
    h2s, dis_col = _tc_a(o, mcol, keep.reshape(npad, 1), degtab, W,
                         first=(li == 0), npad=npad)
    aggparts = _sc_pass(sd64, ew16, h2s, npad, epad, D, ch=64, split=0.80).reshape(NC, npad, D)
    o, z_col = _tc_b1(aggparts, h2s, dis_col, b.reshape(1, D), p.reshape(D, 1), npad)
    m_pack, keep_pack = _tc_b2(z_col.reshape(npad // 128, 128),
                               keep.reshape(npad // 128, 128), kth, npad)
    mcol = m_pack.reshape(npad, 1)
    keep = keep_pack.reshape(npad)

  out, xc = _tc_readout(o, mcol, Wo.reshape(2 * D, 1), bo.reshape(1, 1),
                        float(ks[2]), npad)
  return out, xc
